# baseline (speedup 1.0000x reference)
import jax
import jax.numpy as jnp
from jax.experimental import pallas as pl
from jax.experimental.pallas import tpu as pltpu

_B, _D, _K = 512, 256, 1024
_TB = 8
_EPS = 1e-6
_V = 0.0625


def _bf(t):
    return t.astype(jnp.bfloat16)


def _rq_kernel(x_ref, pq_ref, cb_ref, q_ref, idx_ref, loss_ref, r_scratch):
    step = pl.program_id(0)
    cb = cb_ref[...]

    x_blk = x_ref[...]
    pq_blk = pq_ref[...]

    n = jnp.sqrt(jnp.sum(pq_blk * pq_blk, axis=1, keepdims=True))
    u_blk = pq_blk / jnp.maximum(n, 1e-6)

    eye = jnp.eye(_D, dtype=jnp.float32)

    xc_rows = []
    for t in range(_TB):
        u_row = u_blk[t].reshape(1, _D)
        u_col = u_row.reshape(_D, 1)
        A = (u_col - u_row) * _V
        A2 = jax.lax.dot_general(
            _bf(A), _bf(A), (((1,), (0,)), ((), ())),
            preferred_element_type=jnp.float32)
        c = jnp.sum(u_row * _V)
        Rm = (eye + A) + A2 / (1.0 + c + _EPS)
        r_scratch[t] = Rm
        xr = x_blk[t].reshape(1, _D)
        xc = jax.lax.dot_general(
            _bf(xr), _bf(Rm), (((1,), (0,)), ((), ())),
            preferred_element_type=jnp.float32)
        xc_rows.append(xc)
    xc_blk = jnp.concatenate(xc_rows, axis=0)

    cn2 = jnp.sum(cb * cb, axis=1, keepdims=True)
    dots = jax.lax.dot_general(
        xc_blk, cb, (((1,), (1,)), ((), ())),
        preferred_element_type=jnp.float32,
        precision=jax.lax.Precision.HIGHEST)
    score = cn2.reshape(1, _K) - 2.0 * dots
    mins = jnp.min(score, axis=1, keepdims=True)
    iota_k = jax.lax.broadcasted_iota(jnp.int32, (_TB, _K), 1)
    idx = jnp.min(jnp.where(score <= mins, iota_k, _K), axis=1, keepdims=True)
    idx_ref[...] = idx

    onehot = (iota_k == idx).astype(jnp.float32)
    qc_blk = jax.lax.dot_general(
        onehot, cb, (((1,), (0,)), ((), ())),
        preferred_element_type=jnp.float32,
        precision=jax.lax.Precision.HIGHEST)

    q_rows = []
    for t in range(_TB):
        qc = qc_blk[t].reshape(1, _D)
        q_rows.append(jax.lax.dot_general(
            _bf(qc), _bf(r_scratch[t]), (((1,), (1,)), ((), ())),
            preferred_element_type=jnp.float32))
    q_blk = jnp.concatenate(q_rows, axis=0)
    q_ref[...] = q_blk

    diff = x_blk - q_blk
    part = jnp.sum(diff * diff).reshape(1, 1) * (1.25 / _B)

    @pl.when(step == 0)
    def _():
        loss_ref[...] = jnp.zeros_like(loss_ref)
    loss_ref[...] += part


@jax.jit
def kernel(x, prev_q, codes):
    cb = codes.reshape(_K, _D)
    q, idx, loss = pl.pallas_call(
        _rq_kernel,
        grid=(_B // _TB,),
        in_specs=[
            pl.BlockSpec((_TB, _D), lambda i: (i, 0)),
            pl.BlockSpec((_TB, _D), lambda i: (i, 0)),
            pl.BlockSpec((_K, _D), lambda i: (0, 0)),
        ],
        out_specs=(
            pl.BlockSpec((_TB, _D), lambda i: (i, 0)),
            pl.BlockSpec((_TB, 1), lambda i: (i, 0)),
            pl.BlockSpec((1, 1), lambda i: (0, 0)),
        ),
        out_shape=(
            jax.ShapeDtypeStruct((_B, _D), jnp.float32),
            jax.ShapeDtypeStruct((_B, 1), jnp.int32),
            jax.ShapeDtypeStruct((1, 1), jnp.float32),
        ),
        scratch_shapes=[pltpu.VMEM((_TB, _D, _D), jnp.float32)],
        compiler_params=pltpu.CompilerParams(
            dimension_semantics=("arbitrary",),
        ),
    )(x, prev_q, cb)
    return q, idx.reshape(_B), loss.reshape(())


# speedup vs baseline: 2.7763x; 2.7763x over previous
import jax
import jax.numpy as jnp
from jax.experimental import pallas as pl
from jax.experimental.pallas import tpu as pltpu

_B, _D, _K = 512, 256, 1024
_TB = 32
_EPS = 1e-6
_V = 0.0625


def _bf(t):
    return t.astype(jnp.bfloat16)


def _rq_kernel(x_ref, pq_ref, cb_ref, cbt_ref, q_ref, idx_ref, loss_ref,
               r_scratch, cn2_scratch):
    step = pl.program_id(0)

    @pl.when(step == 0)
    def _():
        cbt = cbt_ref[...]
        cn2_scratch[...] = jnp.sum(cbt * cbt, axis=0, keepdims=True)

    x_blk = x_ref[...]
    pq_blk = pq_ref[...]

    n = jnp.sqrt(jnp.sum(pq_blk * pq_blk, axis=1, keepdims=True))
    u_blk = pq_blk / jnp.maximum(n, 1e-6)
    us_blk = u_blk * _V
    usT = us_blk.T
    c_all = jnp.sum(us_blk, axis=1, keepdims=True)
    denom_all = 1.0 + c_all + _EPS

    eye = jnp.eye(_D, dtype=jnp.float32)

    xc_rows = []
    for t in range(_TB):
        us_row = us_blk[t].reshape(1, _D)
        us_col = usT[:, t].reshape(_D, 1)
        A = us_col - us_row
        Ab = _bf(A)
        A2 = jax.lax.dot_general(
            Ab, Ab, (((1,), (0,)), ((), ())),
            preferred_element_type=jnp.float32)
        Rb = _bf((eye + A) + A2 / denom_all[t, 0])
        r_scratch[t] = Rb
        xr = x_blk[t].reshape(1, _D)
        xc_rows.append(jax.lax.dot_general(
            _bf(xr), Rb, (((1,), (0,)), ((), ())),
            preferred_element_type=jnp.float32))
    xc_blk = jnp.concatenate(xc_rows, axis=0)

    dots = jax.lax.dot_general(
        xc_blk, cbt_ref[...], (((1,), (0,)), ((), ())),
        preferred_element_type=jnp.float32,
        precision=jax.lax.Precision.HIGHEST)
    score = cn2_scratch[...] - 2.0 * dots
    mins = jnp.min(score, axis=1, keepdims=True)
    iota_k = jax.lax.broadcasted_iota(jnp.int32, (_TB, _K), 1)
    idx = jnp.min(jnp.where(score <= mins, iota_k, _K), axis=1, keepdims=True)
    idx_ref[...] = idx

    onehot = (iota_k == idx).astype(jnp.float32)
    qc_blk = jax.lax.dot_general(
        onehot, cb_ref[...], (((1,), (0,)), ((), ())),
        preferred_element_type=jnp.float32,
        precision=jax.lax.Precision.HIGHEST)

    q_rows = []
    for t in range(_TB):
        qc = qc_blk[t].reshape(1, _D)
        q_rows.append(jax.lax.dot_general(
            _bf(qc), r_scratch[t], (((1,), (1,)), ((), ())),
            preferred_element_type=jnp.float32))
    q_blk = jnp.concatenate(q_rows, axis=0)
    q_ref[...] = q_blk

    diff = x_blk - q_blk
    part = jnp.sum(diff * diff).reshape(1, 1) * (1.25 / _B)

    @pl.when(step == 0)
    def _():
        loss_ref[...] = jnp.zeros_like(loss_ref)
    loss_ref[...] += part


@jax.jit
def kernel(x, prev_q, codes):
    cb = codes.reshape(_K, _D)
    cbt = cb.T
    q, idx, loss = pl.pallas_call(
        _rq_kernel,
        grid=(_B // _TB,),
        in_specs=[
            pl.BlockSpec((_TB, _D), lambda i: (i, 0)),
            pl.BlockSpec((_TB, _D), lambda i: (i, 0)),
            pl.BlockSpec((_K, _D), lambda i: (0, 0)),
            pl.BlockSpec((_D, _K), lambda i: (0, 0)),
        ],
        out_specs=(
            pl.BlockSpec((_TB, _D), lambda i: (i, 0)),
            pl.BlockSpec((_TB, 1), lambda i: (i, 0)),
            pl.BlockSpec((1, 1), lambda i: (0, 0)),
        ),
        out_shape=(
            jax.ShapeDtypeStruct((_B, _D), jnp.float32),
            jax.ShapeDtypeStruct((_B, 1), jnp.int32),
            jax.ShapeDtypeStruct((1, 1), jnp.float32),
        ),
        scratch_shapes=[
            pltpu.VMEM((_TB, _D, _D), jnp.bfloat16),
            pltpu.VMEM((1, _K), jnp.float32),
        ],
        compiler_params=pltpu.CompilerParams(
            dimension_semantics=("arbitrary",),
        ),
    )(x, prev_q, cb, cbt)
    return q, idx.reshape(_B), loss.reshape(())


# speedup vs baseline: 4.4807x; 1.6139x over previous
import jax
import jax.numpy as jnp
from jax.experimental import pallas as pl
from jax.experimental.pallas import tpu as pltpu

_B, _D, _K = 512, 256, 1024
_TB = 32
_EPS = 1e-6
_V = 0.0625


def _bf(t):
    return t.astype(jnp.bfloat16)


def _rq_kernel(x_ref, pq_ref, cb_ref, cbt_ref, q_ref, idx_ref, loss_ref,
               r_scratch, cn2_scratch):
    step = pl.program_id(0)

    @pl.when(step == 0)
    def _():
        cbt = cbt_ref[...]
        cn2_scratch[...] = jnp.sum(cbt * cbt, axis=0, keepdims=True)

    x_blk = x_ref[...]
    pq_blk = pq_ref[...]

    n = jnp.sqrt(jnp.sum(pq_blk * pq_blk, axis=1, keepdims=True))
    u_blk = pq_blk / jnp.maximum(n, 1e-6)
    us_blk = u_blk * _V
    usT = us_blk.T
    c_all = jnp.sum(us_blk, axis=1, keepdims=True)
    denom_all = 1.0 + c_all + _EPS

    eye = jnp.eye(_D, dtype=jnp.float32)

    xc_rows = []
    pend = [None] * _TB

    def _issue(t):
        us_row = us_blk[t].reshape(1, _D)
        us_col = usT[:, t].reshape(_D, 1)
        A = us_col - us_row
        Ab = _bf(A)
        A2 = jax.lax.dot_general(
            Ab, Ab, (((1,), (0,)), ((), ())),
            preferred_element_type=jnp.float32)
        pend[t] = (A, A2)

    def _finish(t):
        A, A2 = pend[t]
        pend[t] = None
        Rb = _bf((eye + A) + A2 / denom_all[t, 0])
        r_scratch[t] = Rb
        xr = x_blk[t].reshape(1, _D)
        xc_rows.append(jax.lax.dot_general(
            _bf(xr), Rb, (((1,), (0,)), ((), ())),
            preferred_element_type=jnp.float32))

    _issue(0)
    for t in range(1, _TB):
        _issue(t)
        _finish(t - 1)
    _finish(_TB - 1)
    xc_blk = jnp.concatenate(xc_rows, axis=0)

    dots = jax.lax.dot_general(
        xc_blk, cbt_ref[...], (((1,), (0,)), ((), ())),
        preferred_element_type=jnp.float32,
        precision=jax.lax.Precision.HIGHEST)
    score = cn2_scratch[...] - 2.0 * dots
    mins = jnp.min(score, axis=1, keepdims=True)
    iota_k = jax.lax.broadcasted_iota(jnp.int32, (_TB, _K), 1)
    idx = jnp.min(jnp.where(score <= mins, iota_k, _K), axis=1, keepdims=True)
    idx_ref[...] = idx

    onehot = (iota_k == idx).astype(jnp.float32)
    qc_blk = jax.lax.dot_general(
        onehot, cb_ref[...], (((1,), (0,)), ((), ())),
        preferred_element_type=jnp.float32)

    q_rows = []
    for t in range(_TB):
        qc = qc_blk[t].reshape(1, _D)
        q_rows.append(jax.lax.dot_general(
            _bf(qc), r_scratch[t], (((1,), (1,)), ((), ())),
            preferred_element_type=jnp.float32))
    q_blk = jnp.concatenate(q_rows, axis=0)
    q_ref[...] = q_blk

    diff = x_blk - q_blk
    part = jnp.sum(diff * diff).reshape(1, 1) * (1.25 / _B)

    @pl.when(step == 0)
    def _():
        loss_ref[...] = jnp.zeros_like(loss_ref)
    loss_ref[...] += part


@jax.jit
def kernel(x, prev_q, codes):
    cb = codes.reshape(_K, _D)
    cbt = cb.T
    q, idx, loss = pl.pallas_call(
        _rq_kernel,
        grid=(_B // _TB,),
        in_specs=[
            pl.BlockSpec((_TB, _D), lambda i: (i, 0)),
            pl.BlockSpec((_TB, _D), lambda i: (i, 0)),
            pl.BlockSpec((_K, _D), lambda i: (0, 0)),
            pl.BlockSpec((_D, _K), lambda i: (0, 0)),
        ],
        out_specs=(
            pl.BlockSpec((_TB, _D), lambda i: (i, 0)),
            pl.BlockSpec((_TB, 1), lambda i: (i, 0)),
            pl.BlockSpec((1, 1), lambda i: (0, 0)),
        ),
        out_shape=(
            jax.ShapeDtypeStruct((_B, _D), jnp.float32),
            jax.ShapeDtypeStruct((_B, 1), jnp.int32),
            jax.ShapeDtypeStruct((1, 1), jnp.float32),
        ),
        scratch_shapes=[
            pltpu.VMEM((_TB, _D, _D), jnp.bfloat16),
            pltpu.VMEM((1, _K), jnp.float32),
        ],
        compiler_params=pltpu.CompilerParams(
            dimension_semantics=("arbitrary",),
        ),
    )(x, prev_q, cb, cbt)
    return q, idx.reshape(_B), loss.reshape(())


# speedup vs baseline: 4.8266x; 1.0772x over previous
import jax
import jax.numpy as jnp
from jax.experimental import pallas as pl
from jax.experimental.pallas import tpu as pltpu

_B, _D, _K = 512, 256, 1024
_TB = 64
_H = 32
_EPS = 1e-6
_V = 0.0625


def _bf(t):
    return t.astype(jnp.bfloat16)


def _rq_kernel(x_ref, pq_ref, cb_ref, cbt_ref, q_ref, idx_ref, loss_ref,
               r_scratch, cn2_scratch):
    step = pl.program_id(0)

    @pl.when(step == 0)
    def _():
        cbt = cbt_ref[...]
        cn2_scratch[...] = jnp.sum(cbt * cbt, axis=0, keepdims=True)

    x_blk = x_ref[...]
    pq_blk = pq_ref[...]

    n = jnp.sqrt(jnp.sum(pq_blk * pq_blk, axis=1, keepdims=True))
    u_blk = pq_blk / jnp.maximum(n, 1e-6)
    us_blk = u_blk * _V
    usT = us_blk.T
    c_all = jnp.sum(us_blk, axis=1, keepdims=True)
    denom_all = 1.0 + c_all + _EPS

    eye = jnp.eye(_D, dtype=jnp.float32)

    pend = [None] * _TB

    def _issue(t):
        us_row = us_blk[t].reshape(1, _D)
        us_col = usT[:, t].reshape(_D, 1)
        A = us_col - us_row
        A2 = jax.lax.dot_general(
            A, A, (((1,), (0,)), ((), ())),
            preferred_element_type=jnp.float32)
        pend[t] = (A, A2)

    def _finish(t, xc_rows):
        A, A2 = pend[t]
        pend[t] = None
        Rb = _bf((eye + A) + A2 / denom_all[t, 0])
        r_scratch[t] = Rb
        xr = x_blk[t].reshape(1, _D)
        xc_rows.append(jax.lax.dot_general(
            _bf(xr), Rb, (((1,), (0,)), ((), ())),
            preferred_element_type=jnp.float32))

    def _rotate_half(h):
        xc_rows = []
        base = h * _H
        _issue(base)
        for t in range(base + 1, base + _H):
            _issue(t)
            _finish(t - 1, xc_rows)
        _finish(base + _H - 1, xc_rows)
        return jnp.concatenate(xc_rows, axis=0)

    def _dist_half(xc_half):
        dots = jax.lax.dot_general(
            xc_half, cbt_ref[...], (((1,), (0,)), ((), ())),
            preferred_element_type=jnp.float32,
            precision=jax.lax.Precision.HIGHEST)
        return cn2_scratch[...] - 2.0 * dots

    def _tail_half(h, score):
        base = h * _H
        mins = jnp.min(score, axis=1, keepdims=True)
        iota_k = jax.lax.broadcasted_iota(jnp.int32, (_H, _K), 1)
        idx = jnp.min(jnp.where(score <= mins, iota_k, _K),
                      axis=1, keepdims=True)
        idx_ref[base:base + _H, :] = idx

        onehot = (iota_k == idx).astype(jnp.float32)
        qc_half = jax.lax.dot_general(
            onehot, cb_ref[...], (((1,), (0,)), ((), ())),
            preferred_element_type=jnp.float32)

        q_rows = []
        for t in range(_H):
            qc = qc_half[t].reshape(1, _D)
            q_rows.append(jax.lax.dot_general(
                _bf(qc), r_scratch[base + t], (((1,), (1,)), ((), ())),
                preferred_element_type=jnp.float32))
        q_half = jnp.concatenate(q_rows, axis=0)
        q_ref[base:base + _H, :] = q_half
        diff = x_blk[base:base + _H, :] - q_half
        return jnp.sum(diff * diff)

    n_chunks = _TB // _H
    scores = [None] * n_chunks
    losses = []
    scores[0] = _dist_half(_rotate_half(0))
    for h in range(1, n_chunks):
        scores[h] = _dist_half(_rotate_half(h))
        losses.append(_tail_half(h - 1, scores[h - 1]))
        scores[h - 1] = None
    losses.append(_tail_half(n_chunks - 1, scores[n_chunks - 1]))
    total = losses[0]
    for l in losses[1:]:
        total = total + l
    part = total.reshape(1, 1) * (1.25 / _B)

    @pl.when(step == 0)
    def _():
        loss_ref[...] = jnp.zeros_like(loss_ref)
    loss_ref[...] += part


@jax.jit
def kernel(x, prev_q, codes):
    cb = codes.reshape(_K, _D)
    cbt = cb.T
    q, idx, loss = pl.pallas_call(
        _rq_kernel,
        grid=(_B // _TB,),
        in_specs=[
            pl.BlockSpec((_TB, _D), lambda i: (i, 0)),
            pl.BlockSpec((_TB, _D), lambda i: (i, 0)),
            pl.BlockSpec((_K, _D), lambda i: (0, 0)),
            pl.BlockSpec((_D, _K), lambda i: (0, 0)),
        ],
        out_specs=(
            pl.BlockSpec((_TB, _D), lambda i: (i, 0)),
            pl.BlockSpec((_TB, 1), lambda i: (i, 0)),
            pl.BlockSpec((1, 1), lambda i: (0, 0)),
        ),
        out_shape=(
            jax.ShapeDtypeStruct((_B, _D), jnp.float32),
            jax.ShapeDtypeStruct((_B, 1), jnp.int32),
            jax.ShapeDtypeStruct((1, 1), jnp.float32),
        ),
        scratch_shapes=[
            pltpu.VMEM((_TB, _D, _D), jnp.bfloat16),
            pltpu.VMEM((1, _K), jnp.float32),
        ],
        compiler_params=pltpu.CompilerParams(
            dimension_semantics=("arbitrary",),
        ),
    )(x, prev_q, cb, cbt)
    return q, idx.reshape(_B), loss.reshape(())
